# Initial kernel scaffold; baseline (speedup 1.0000x reference)
import jax
import jax.numpy as jnp
from jax.experimental import pallas as pl


def kernel(positions, atomic_numbers, edge_index, edge_lengths, edge_vectors, W):
    raise NotImplementedError("write your pallas kernel here")


# speedup vs baseline: 23.9842x; 23.9842x over previous
import functools
import numpy as np
import jax
import jax.numpy as jnp
from jax import lax
from jax.experimental import pallas as pl
from jax.experimental.pallas import tpu as pltpu
from jax.experimental.pallas import tpu_sc as plsc

_ZS = (1, 6, 7, 8)
_CUTOFF = 5.5
_N = 10000
_E = 160000

_NT = 10240
_EPAD = 163840
_EPW = _EPAD // 16
_BBLK = 128
_NBLK = _EPW // _BBLK
_ROWS_PER_SUB = _NT // 16
_PM = 30
_SCW = 96


def _build_consts():
    r1 = np.zeros((192, 540), np.float32)
    r2 = np.zeros((8, 540), np.float32)
    for j in range(6):
        for k in range(10):
            m = k * 6 + j
            scol = (m // _PM) * _SCW + (m % _PM)
            for a in range(3):
                for b in range(3):
                    c = (j * 10 + k) * 9 + a * 3 + b
                    r1[scol + a * 32, c] = 1.0
                    r2[b, c] = 1.0
    rb = np.zeros((1080, 162), np.float32)
    l2_pref = {4: 1.0, 5: 2.0, 6: 2.0, 7: 1.0, 8: 2.0, 9: 1.0}
    for j in range(6):
        for c9 in range(9):
            rb[(j * 10 + 0) * 9 + c9, j * 27 + 0 * 9 + c9] = 1.0
            for k in (1, 2, 3):
                rb[540 + (j * 10 + k) * 9 + c9, j * 27 + 1 * 9 + c9] = 1.0
            for k, pref in l2_pref.items():
                rb[540 + (j * 10 + k) * 9 + c9, j * 27 + 2 * 9 + c9] = pref
    return r1, r2, rb


_R1, _R2, _RB = _build_consts()


def _embed_body(an_ref, w_ref, out_ref):
    an0 = an_ref[0:1, :]
    rows = []
    for a in range(3):
        acc = jnp.zeros(an0.shape, jnp.float32)
        for zi, z in enumerate(_ZS):
            acc = acc + jnp.where(an0 == z, w_ref[zi, a], 0.0)
        rows.append(acc)
    rows.append(jnp.zeros((5, an0.shape[1]), jnp.float32))
    out_ref[...] = jnp.concatenate(rows, axis=0)


def _embed(an8, w):
    return pl.pallas_call(
        _embed_body,
        out_shape=jax.ShapeDtypeStruct((8, _NT), jnp.float32),
        in_specs=[
            pl.BlockSpec(memory_space=pltpu.VMEM),
            pl.BlockSpec(memory_space=pltpu.SMEM),
        ],
        out_specs=pl.BlockSpec(memory_space=pltpu.VMEM),
    )(an8, w)


_BE = 512


def _edge_body(rv_ref, out_ref):
    r = rv_ref[0:1, :]
    vx = rv_ref[1:2, :]
    vy = rv_ref[2:3, :]
    vz = rv_ref[3:4, :]
    theta = (np.pi / _CUTOFF) * r
    s1 = jnp.sin(theta)
    c1 = jnp.cos(theta)
    sins = [s1, 2.0 * c1 * s1]
    for _ in range(4):
        sins.append(2.0 * c1 * sins[-1] - sins[-2])
    x = r * (1.0 / _CUTOFF)
    x2 = x * x
    x3 = x2 * x
    x6 = x3 * x3
    fc = 1.0 - 28.0 * x6 + 48.0 * x6 * x - 21.0 * x6 * x2
    fc = jnp.where(x < 1.0, fc, 0.0)
    pref = np.float32(np.sqrt(2.0 / _CUTOFF)) * fc / r
    rad = [pref * s for s in sins]
    one = jnp.ones(r.shape, jnp.float32)
    ang = [one, vx, vy, vz, vx * vx, vx * vy, vx * vz, vy * vy, vy * vz,
           vz * vz]
    pad2 = jnp.zeros((2, r.shape[1]), jnp.float32)
    rows = []
    for part in range(2):
        for mm in range(_PM):
            m = part * _PM + mm
            rows.append(rad[m % 6] * ang[m // 6])
        rows.append(pad2)
    out_ref[...] = jnp.concatenate(rows, axis=0)


def _edgefeat(rv):
    return pl.pallas_call(
        _edge_body,
        grid=(_EPAD // _BE,),
        out_shape=jax.ShapeDtypeStruct((64, _EPAD), jnp.float32),
        in_specs=[pl.BlockSpec((8, _BE), lambda i: (0, i))],
        out_specs=pl.BlockSpec((64, _BE), lambda i: (0, i)),
    )(rv)


def _sc_body(pt_hbm, src_hbm, dst_hbm, ex_hbm, ey_hbm, ez_hbm, out_hbm,
             srcv, dstv, pv, gbuf, embx, emby, embz, s_sh):
    cid = lax.axis_index("c")
    sid = lax.axis_index("s")
    base = sid * _EPW
    prow = cid * 32

    pltpu.sync_copy(ex_hbm, embx)
    pltpu.sync_copy(ey_hbm, emby)
    pltpu.sync_copy(ez_hbm, embz)

    zero16 = jnp.zeros((16,), jnp.float32)

    def _zrow(i, c):
        for cc in range(_SCW // 16):
            gbuf[i, pl.ds(cc * 16, 16)] = zero16
        return c

    lax.fori_loop(0, _BBLK, _zrow, 0)
    for t in range(_ROWS_PER_SUB // _BBLK):
        pltpu.sync_copy(gbuf, s_sh.at[pl.ds(sid * _ROWS_PER_SUB + t * _BBLK,
                                            _BBLK)])
    plsc.subcore_barrier()

    lane = lax.iota(jnp.int32, 16)

    def _blk(blk, c):
        ebase = base + blk * _BBLK
        pltpu.sync_copy(src_hbm.at[pl.ds(ebase, _BBLK)], srcv)
        pltpu.sync_copy(dst_hbm.at[pl.ds(ebase, _BBLK)], dstv)
        pltpu.sync_copy(pt_hbm.at[pl.ds(prow, 32), pl.ds(ebase, _BBLK)], pv)
        for g in range(_BBLK // 16):
            s16 = srcv[pl.ds(g * 16, 16)]
            ex = plsc.load_gather(embx, [s16])
            ey = plsc.load_gather(emby, [s16])
            ez = plsc.load_gather(embz, [s16])
            e16 = lane + (g * 16)
            for mm in range(_PM):
                pm = pv[mm, pl.ds(g * 16, 16)]
                for a, ev in ((0, ex), (1, ey), (2, ez)):
                    col = jnp.full((16,), a * 32 + mm, jnp.int32)
                    plsc.store_scatter(gbuf, [e16, col], pm * ev)
        pltpu.sync_copy(gbuf, s_sh.at[dstv], add=True)
        return c

    lax.fori_loop(0, _NBLK, _blk, 0)
    plsc.subcore_barrier()

    for t in range(_ROWS_PER_SUB // _BBLK):
        r0 = sid * _ROWS_PER_SUB + t * _BBLK
        pltpu.sync_copy(s_sh.at[pl.ds(r0, _BBLK)], gbuf)
        pltpu.sync_copy(gbuf, out_hbm.at[cid, pl.ds(r0, _BBLK)])


_sc_call = functools.partial(
    pl.kernel,
    out_type=pltpu.HBM((2, _NT, _SCW), jnp.float32),
    mesh=plsc.VectorSubcoreMesh(core_axis_name="c", subcore_axis_name="s"),
    compiler_params=pltpu.CompilerParams(needs_layout_passes=False,
                                         use_tc_tiling_on_sc=False),
    scratch_types=[
        pltpu.VMEM((_BBLK,), jnp.int32),
        pltpu.VMEM((_BBLK,), jnp.int32),
        pltpu.VMEM((32, _BBLK), jnp.float32),
        pltpu.VMEM((_BBLK, _SCW), jnp.float32),
        pltpu.VMEM((_NT,), jnp.float32),
        pltpu.VMEM((_NT,), jnp.float32),
        pltpu.VMEM((_NT,), jnp.float32),
        pltpu.VMEM_SHARED((_NT, _SCW), jnp.float32),
    ],
)(_sc_body)


_BN = 256


def _combine_body(s0_ref, s1_ref, emb_ref, r1_ref, r2_ref, rb_ref,
                  a_ref, b_ref):
    s = jnp.concatenate([s0_ref[...], s1_ref[...]], axis=1)
    f = lax.dot_general(emb_ref[...], r2_ref[...], (((0,), (0,)), ((), ())),
                        preferred_element_type=jnp.float32)
    a = jnp.dot(s, r1_ref[...], preferred_element_type=jnp.float32) * f
    a_ref[...] = a
    aa = jnp.concatenate([a, a * a], axis=1)
    b_ref[...] = jnp.dot(aa, rb_ref[...], preferred_element_type=jnp.float32)


def _combine(s0, s1, emb_cols, r1, r2, rb):
    return pl.pallas_call(
        _combine_body,
        grid=(_NT // _BN,),
        out_shape=(
            jax.ShapeDtypeStruct((_NT, 540), jnp.float32),
            jax.ShapeDtypeStruct((_NT, 162), jnp.float32),
        ),
        in_specs=[
            pl.BlockSpec((_BN, _SCW), lambda i: (i, 0)),
            pl.BlockSpec((_BN, _SCW), lambda i: (i, 0)),
            pl.BlockSpec((8, _BN), lambda i: (0, i)),
            pl.BlockSpec((192, 540), lambda i: (0, 0)),
            pl.BlockSpec((8, 540), lambda i: (0, 0)),
            pl.BlockSpec((1080, 162), lambda i: (0, 0)),
        ],
        out_specs=(
            pl.BlockSpec((_BN, 540), lambda i: (i, 0)),
            pl.BlockSpec((_BN, 162), lambda i: (i, 0)),
        ),
    )(s0, s1, emb_cols, r1, r2, rb)


def kernel(positions, atomic_numbers, edge_index, edge_lengths, edge_vectors,
           W):
    src = edge_index[0].astype(jnp.int32)
    dst = edge_index[1].astype(jnp.int32)
    pad_e = _EPAD - _E
    src_p = jnp.concatenate([src, jnp.zeros((pad_e,), jnp.int32)])
    dst_p = jnp.concatenate([dst, jnp.full((pad_e,), _N, jnp.int32)])

    rv = jnp.zeros((8, _EPAD), jnp.float32)
    rv = rv.at[0, :_E].set(edge_lengths[:, 0]).at[0, _E:].set(1.0)
    rv = rv.at[1:4, :_E].set(edge_vectors.T)

    an8 = jnp.zeros((8, _NT), jnp.int32).at[0, :_N].set(
        atomic_numbers.astype(jnp.int32))

    emb_cols = _embed(an8, W.astype(jnp.float32))
    pt = _edgefeat(rv)
    s2 = _sc_call(pt, src_p, dst_p, emb_cols[0], emb_cols[1], emb_cols[2])
    a_pad, b_pad = _combine(s2[0], s2[1], emb_cols,
                            jnp.asarray(_R1), jnp.asarray(_R2),
                            jnp.asarray(_RB))
    node_a = a_pad[:_N].reshape(_N, 6, 10, 9)
    node_b = b_pad[:_N].reshape(_N, 6, 3, 9)
    return node_a, node_b


# speedup vs baseline: 27.5126x; 1.1471x over previous
import functools
import numpy as np
import jax
import jax.numpy as jnp
from jax import lax
from jax.experimental import pallas as pl
from jax.experimental.pallas import tpu as pltpu
from jax.experimental.pallas import tpu_sc as plsc

_ZS = (1, 6, 7, 8)
_CUTOFF = 5.5
_N = 10000
_E = 160000

_NT = 10240
_EPAD = 163840
_EPW = _EPAD // 16
_BBLK = 128
_NBLK = _EPW // _BBLK
_ROWS_PER_SUB = _NT // 16
_PM = 30
_SCW = 96


def _build_consts():
    r1 = np.zeros((192, 540), np.float32)
    r2 = np.zeros((8, 540), np.float32)
    for j in range(6):
        for k in range(10):
            m = k * 6 + j
            scol = (m // _PM) * _SCW + (m % _PM)
            for a in range(3):
                for b in range(3):
                    c = (j * 10 + k) * 9 + a * 3 + b
                    r1[scol + a * 32, c] = 1.0
                    r2[b, c] = 1.0
    rb = np.zeros((1080, 162), np.float32)
    l2_pref = {4: 1.0, 5: 2.0, 6: 2.0, 7: 1.0, 8: 2.0, 9: 1.0}
    for j in range(6):
        for c9 in range(9):
            rb[(j * 10 + 0) * 9 + c9, j * 27 + 0 * 9 + c9] = 1.0
            for k in (1, 2, 3):
                rb[540 + (j * 10 + k) * 9 + c9, j * 27 + 1 * 9 + c9] = 1.0
            for k, pref in l2_pref.items():
                rb[540 + (j * 10 + k) * 9 + c9, j * 27 + 2 * 9 + c9] = pref
    return r1, r2, rb


_R1, _R2, _RB = _build_consts()


def _embed_body(an_ref, w_ref, out_ref):
    an0 = an_ref[0:1, :]
    rows = []
    for a in range(3):
        acc = jnp.zeros(an0.shape, jnp.float32)
        for zi, z in enumerate(_ZS):
            acc = acc + jnp.where(an0 == z, w_ref[zi, a], 0.0)
        rows.append(acc)
    rows.append(jnp.zeros((5, an0.shape[1]), jnp.float32))
    out_ref[...] = jnp.concatenate(rows, axis=0)


def _embed(an8, w):
    return pl.pallas_call(
        _embed_body,
        out_shape=jax.ShapeDtypeStruct((8, _NT), jnp.float32),
        in_specs=[
            pl.BlockSpec(memory_space=pltpu.VMEM),
            pl.BlockSpec(memory_space=pltpu.SMEM),
        ],
        out_specs=pl.BlockSpec(memory_space=pltpu.VMEM),
    )(an8, w)


_BE = 512


def _edge_body(rv_ref, out_ref):
    r = rv_ref[0:1, :]
    vx = rv_ref[1:2, :]
    vy = rv_ref[2:3, :]
    vz = rv_ref[3:4, :]
    theta = (np.pi / _CUTOFF) * r
    s1 = jnp.sin(theta)
    c1 = jnp.cos(theta)
    sins = [s1, 2.0 * c1 * s1]
    for _ in range(4):
        sins.append(2.0 * c1 * sins[-1] - sins[-2])
    x = r * (1.0 / _CUTOFF)
    x2 = x * x
    x3 = x2 * x
    x6 = x3 * x3
    fc = 1.0 - 28.0 * x6 + 48.0 * x6 * x - 21.0 * x6 * x2
    fc = jnp.where(x < 1.0, fc, 0.0)
    pref = np.float32(np.sqrt(2.0 / _CUTOFF)) * fc / r
    rad = [pref * s for s in sins]
    one = jnp.ones(r.shape, jnp.float32)
    ang = [one, vx, vy, vz, vx * vx, vx * vy, vx * vz, vy * vy, vy * vz,
           vz * vz]
    pad2 = jnp.zeros((2, r.shape[1]), jnp.float32)
    rows = []
    for part in range(2):
        for mm in range(_PM):
            m = part * _PM + mm
            rows.append(rad[m % 6] * ang[m // 6])
        rows.append(pad2)
    out_ref[...] = jnp.concatenate(rows, axis=0)


def _edgefeat(rv):
    return pl.pallas_call(
        _edge_body,
        grid=(_EPAD // _BE,),
        out_shape=jax.ShapeDtypeStruct((64, _EPAD), jnp.float32),
        in_specs=[pl.BlockSpec((8, _BE), lambda i: (0, i))],
        out_specs=pl.BlockSpec((64, _BE), lambda i: (0, i)),
    )(rv)


def _sc_body(pt_hbm, src_hbm, dst_hbm, emb_hbm, out_hbm,
             srcv, dstv, pv, gbuf, embx, emby, embz, s_sh):
    cid = lax.axis_index("c")
    sid = lax.axis_index("s")
    base = sid * _EPW
    prow = cid * 32

    pltpu.sync_copy(emb_hbm.at[0], embx)
    pltpu.sync_copy(emb_hbm.at[1], emby)
    pltpu.sync_copy(emb_hbm.at[2], embz)

    zero16 = jnp.zeros((16,), jnp.float32)

    def _zrow(i, c):
        for cc in range(_SCW // 16):
            gbuf[i, pl.ds(cc * 16, 16)] = zero16
        return c

    lax.fori_loop(0, _BBLK, _zrow, 0)
    for t in range(_ROWS_PER_SUB // _BBLK):
        pltpu.sync_copy(gbuf, s_sh.at[pl.ds(sid * _ROWS_PER_SUB + t * _BBLK,
                                            _BBLK)])
    plsc.subcore_barrier()

    lane = lax.iota(jnp.int32, 16)

    def _blk(blk, c):
        ebase = base + blk * _BBLK
        pltpu.sync_copy(src_hbm.at[pl.ds(ebase, _BBLK)], srcv)
        pltpu.sync_copy(dst_hbm.at[pl.ds(ebase, _BBLK)], dstv)
        pltpu.sync_copy(pt_hbm.at[pl.ds(prow, 32), pl.ds(ebase, _BBLK)], pv)
        for g in range(_BBLK // 16):
            s16 = srcv[pl.ds(g * 16, 16)]
            ex = plsc.load_gather(embx, [s16])
            ey = plsc.load_gather(emby, [s16])
            ez = plsc.load_gather(embz, [s16])
            e16 = lane + (g * 16)
            for mm in range(_PM):
                pm = pv[mm, pl.ds(g * 16, 16)]
                for a, ev in ((0, ex), (1, ey), (2, ez)):
                    col = jnp.full((16,), a * 32 + mm, jnp.int32)
                    plsc.store_scatter(gbuf, [e16, col], pm * ev)
        pltpu.sync_copy(gbuf, s_sh.at[dstv], add=True)
        return c

    lax.fori_loop(0, _NBLK, _blk, 0)
    plsc.subcore_barrier()

    for t in range(_ROWS_PER_SUB // _BBLK):
        r0 = sid * _ROWS_PER_SUB + t * _BBLK
        pltpu.sync_copy(s_sh.at[pl.ds(r0, _BBLK)], gbuf)
        pltpu.sync_copy(gbuf, out_hbm.at[cid, pl.ds(r0, _BBLK)])


_sc_call = functools.partial(
    pl.kernel,
    out_type=pltpu.HBM((2, _NT, _SCW), jnp.float32),
    mesh=plsc.VectorSubcoreMesh(core_axis_name="c", subcore_axis_name="s"),
    compiler_params=pltpu.CompilerParams(needs_layout_passes=False,
                                         use_tc_tiling_on_sc=False),
    scratch_types=[
        pltpu.VMEM((_BBLK,), jnp.int32),
        pltpu.VMEM((_BBLK,), jnp.int32),
        pltpu.VMEM((32, _BBLK), jnp.float32),
        pltpu.VMEM((_BBLK, _SCW), jnp.float32),
        pltpu.VMEM((_NT,), jnp.float32),
        pltpu.VMEM((_NT,), jnp.float32),
        pltpu.VMEM((_NT,), jnp.float32),
        pltpu.VMEM_SHARED((_NT, _SCW), jnp.float32),
    ],
)(_sc_body)


_BN = 256


def _combine_body(s0_ref, s1_ref, emb_ref, r1_ref, r2_ref, rb_ref,
                  a_ref, b_ref):
    s = jnp.concatenate([s0_ref[0], s1_ref[0]], axis=1)
    f = lax.dot_general(emb_ref[...], r2_ref[...], (((0,), (0,)), ((), ())),
                        preferred_element_type=jnp.float32)
    a = jnp.dot(s, r1_ref[...], preferred_element_type=jnp.float32) * f
    a_ref[...] = a
    aa = jnp.concatenate([a, a * a], axis=1)
    b_ref[...] = jnp.dot(aa, rb_ref[...], preferred_element_type=jnp.float32)


def _combine(s2, emb_cols, r1, r2, rb):
    return pl.pallas_call(
        _combine_body,
        grid=(_NT // _BN,),
        out_shape=(
            jax.ShapeDtypeStruct((_N, 540), jnp.float32),
            jax.ShapeDtypeStruct((_N, 162), jnp.float32),
        ),
        in_specs=[
            pl.BlockSpec((1, _BN, _SCW), lambda i: (0, i, 0)),
            pl.BlockSpec((1, _BN, _SCW), lambda i: (1, i, 0)),
            pl.BlockSpec((8, _BN), lambda i: (0, i)),
            pl.BlockSpec((192, 540), lambda i: (0, 0)),
            pl.BlockSpec((8, 540), lambda i: (0, 0)),
            pl.BlockSpec((1080, 162), lambda i: (0, 0)),
        ],
        out_specs=(
            pl.BlockSpec((_BN, 540), lambda i: (i, 0)),
            pl.BlockSpec((_BN, 162), lambda i: (i, 0)),
        ),
    )(s2, s2, emb_cols, r1, r2, rb)


def kernel(positions, atomic_numbers, edge_index, edge_lengths, edge_vectors,
           W):
    src = edge_index[0].astype(jnp.int32)
    dst = edge_index[1].astype(jnp.int32)
    pad_e = _EPAD - _E
    src_p = jnp.concatenate([src, jnp.zeros((pad_e,), jnp.int32)])
    dst_p = jnp.concatenate([dst, jnp.full((pad_e,), _N, jnp.int32)])

    rv = jnp.concatenate([edge_lengths.reshape(1, _E),
                          edge_vectors.T.astype(jnp.float32),
                          jnp.zeros((4, _E), jnp.float32)], axis=0)
    rv = jnp.pad(rv, ((0, 0), (0, pad_e)), constant_values=1.0)

    an8 = jnp.pad(atomic_numbers.astype(jnp.int32).reshape(1, _N),
                  ((0, 7), (0, _NT - _N)))

    emb_cols = _embed(an8, W.astype(jnp.float32))
    pt = _edgefeat(rv)
    s2 = _sc_call(pt, src_p, dst_p, emb_cols)
    node_a, node_b = _combine(s2, emb_cols,
                              jnp.asarray(_R1), jnp.asarray(_R2),
                              jnp.asarray(_RB))
    return node_a.reshape(_N, 6, 10, 9), node_b.reshape(_N, 6, 3, 9)


# speedup vs baseline: 33.4658x; 1.2164x over previous
import functools
import numpy as np
import jax
import jax.numpy as jnp
from jax import lax
from jax.experimental import pallas as pl
from jax.experimental.pallas import tpu as pltpu
from jax.experimental.pallas import tpu_sc as plsc

_ZS = (1, 6, 7, 8)
_CUTOFF = 5.5
_N = 10000
_E = 160000

_NT = 10240
_EPAD = 163840
_BSZ = 1024
_NBLK = _EPAD // _BSZ
_ROWS_PER_SUB = _NT // 16
_PM = 30
_SCW = 96


def _build_consts():
    r1 = np.zeros((256, 540), np.float32)
    r2 = np.zeros((8, 540), np.float32)
    for j in range(6):
        for k in range(10):
            m = k * 6 + j
            part, mm = divmod(m, _PM)
            for a in range(3):
                scol = part * 128 + (mm // 2) * 8 + a * 2 + (mm % 2)
                for b in range(3):
                    c = (j * 10 + k) * 9 + a * 3 + b
                    r1[scol, c] = 1.0
                    r2[b, c] = 1.0
    rb = np.zeros((1080, 162), np.float32)
    l2_pref = {4: 1.0, 5: 2.0, 6: 2.0, 7: 1.0, 8: 2.0, 9: 1.0}
    for j in range(6):
        for c9 in range(9):
            rb[(j * 10 + 0) * 9 + c9, j * 27 + 0 * 9 + c9] = 1.0
            for k in (1, 2, 3):
                rb[540 + (j * 10 + k) * 9 + c9, j * 27 + 1 * 9 + c9] = 1.0
            for k, pref in l2_pref.items():
                rb[540 + (j * 10 + k) * 9 + c9, j * 27 + 2 * 9 + c9] = pref
    return r1, r2, rb


_R1, _R2, _RB = _build_consts()


def _embed_body(an_ref, w_ref, out_ref):
    an0 = an_ref[0:1, :]
    rows = []
    for a in range(3):
        acc = jnp.zeros(an0.shape, jnp.float32)
        for zi, z in enumerate(_ZS):
            acc = acc + jnp.where(an0 == z, w_ref[zi, a], 0.0)
        rows.append(acc)
    rows.append(jnp.zeros((5, an0.shape[1]), jnp.float32))
    out_ref[...] = jnp.concatenate(rows, axis=0)


def _embed(an8, w):
    return pl.pallas_call(
        _embed_body,
        out_shape=jax.ShapeDtypeStruct((8, _NT), jnp.float32),
        in_specs=[
            pl.BlockSpec(memory_space=pltpu.VMEM),
            pl.BlockSpec(memory_space=pltpu.SMEM),
        ],
        out_specs=pl.BlockSpec(memory_space=pltpu.VMEM),
    )(an8, w)


_BE = 512


def _edge_body(rv_ref, out_ref):
    r = rv_ref[0:1, :]
    vx = rv_ref[1:2, :]
    vy = rv_ref[2:3, :]
    vz = rv_ref[3:4, :]
    theta = (np.pi / _CUTOFF) * r
    s1 = jnp.sin(theta)
    c1 = jnp.cos(theta)
    sins = [s1, 2.0 * c1 * s1]
    for _ in range(4):
        sins.append(2.0 * c1 * sins[-1] - sins[-2])
    x = r * (1.0 / _CUTOFF)
    x2 = x * x
    x3 = x2 * x
    x6 = x3 * x3
    fc = 1.0 - 28.0 * x6 + 48.0 * x6 * x - 21.0 * x6 * x2
    fc = jnp.where(x < 1.0, fc, 0.0)
    pref = np.float32(np.sqrt(2.0 / _CUTOFF)) * fc / r
    rad = [pref * s for s in sins]
    one = jnp.ones(r.shape, jnp.float32)
    ang = [one, vx, vy, vz, vx * vx, vx * vy, vx * vz, vy * vy, vy * vz,
           vz * vz]
    zrow = jnp.zeros((1, r.shape[1]), jnp.float32)
    planes = []
    for r2 in range(32):
        cid, sid = divmod(r2, 16)
        pair = []
        for q in range(2):
            m = cid * _PM + 2 * sid + q
            pair.append(rad[m % 6] * ang[m // 6] if m % _PM == 2 * sid + q
                        and 2 * sid + q < _PM else zrow)
        planes.append(jnp.concatenate(pair, axis=0)[None])
    out_ref[...] = jnp.concatenate(planes, axis=0)


def _edgefeat(rv):
    return pl.pallas_call(
        _edge_body,
        grid=(_EPAD // _BE,),
        out_shape=jax.ShapeDtypeStruct((32, 2, _EPAD), jnp.float32),
        in_specs=[pl.BlockSpec((8, _BE), lambda i: (0, i))],
        out_specs=pl.BlockSpec((32, 2, _BE), lambda i: (0, 0, i)),
    )(rv)


def _sc_body(pt_hbm, src_hbm, dst_hbm, emb_hbm, out_hbm,
             srcA, dstA, pvA, srcB, dstB, pvB, acc,
             embx, emby, embz, semA, semB):
    cid = lax.axis_index("c")
    sid = lax.axis_index("s")
    r2 = cid * 16 + sid

    pltpu.sync_copy(emb_hbm.at[0], embx)
    pltpu.sync_copy(emb_hbm.at[1], emby)
    pltpu.sync_copy(emb_hbm.at[2], embz)

    zero16 = jnp.zeros((16,), jnp.float32)
    lane = lax.iota(jnp.int32, 16)

    def _zrow(i, c):
        r16 = i * 16 + lane
        for cc in range(8):
            plsc.store_scatter(acc, [r16, jnp.full((16,), cc, jnp.int32)],
                               zero16)
        return c

    lax.fori_loop(0, _NT // 16, _zrow, 0)

    def _fire(b, sbuf, dbuf, pbuf, sem):
        eb = jnp.minimum(b, _NBLK - 1) * _BSZ
        pltpu.async_copy(src_hbm.at[pl.ds(eb, _BSZ)], sbuf, sem)
        pltpu.async_copy(dst_hbm.at[pl.ds(eb, _BSZ)], dbuf, sem)
        pltpu.async_copy(pt_hbm.at[r2, :, pl.ds(eb, _BSZ)], pbuf, sem)

    def _wait(sbuf, dbuf, pbuf, sem):
        pltpu.make_async_copy(src_hbm.at[pl.ds(0, _BSZ)], sbuf, sem).wait()
        pltpu.make_async_copy(dst_hbm.at[pl.ds(0, _BSZ)], dbuf, sem).wait()
        pltpu.make_async_copy(pt_hbm.at[0, :, pl.ds(0, _BSZ)], pbuf,
                              sem).wait()

    def _compute(sbuf, dbuf, pbuf):
        for g in range(_BSZ // 16):
            sl = pl.ds(g * 16, 16)
            s16 = sbuf[sl]
            d16 = dbuf[sl]
            ex = plsc.load_gather(embx, [s16])
            ey = plsc.load_gather(emby, [s16])
            ez = plsc.load_gather(embz, [s16])
            pm0 = pbuf[0, sl]
            pm1 = pbuf[1, sl]
            for q, pm in ((0, pm0), (1, pm1)):
                for a, ev in ((0, ex), (1, ey), (2, ez)):
                    col = jnp.full((16,), a * 2 + q, jnp.int32)
                    plsc.addupdate_scatter(acc, [d16, col], pm * ev)

    _fire(0, srcA, dstA, pvA, semA)

    def _body2(i, c):
        b0 = i * 2
        _fire(b0 + 1, srcB, dstB, pvB, semB)
        _wait(srcA, dstA, pvA, semA)
        _compute(srcA, dstA, pvA)
        _fire(b0 + 2, srcA, dstA, pvA, semA)
        _wait(srcB, dstB, pvB, semB)
        _compute(srcB, dstB, pvB)
        return c

    lax.fori_loop(0, _NBLK // 2, _body2, 0)
    _wait(srcA, dstA, pvA, semA)

    pltpu.sync_copy(acc, out_hbm.at[cid, sid])


_sc_call = functools.partial(
    pl.kernel,
    out_type=pltpu.HBM((2, 16, _NT, 8), jnp.float32),
    mesh=plsc.VectorSubcoreMesh(core_axis_name="c", subcore_axis_name="s",
                                num_cores=2, num_subcores=16),
    compiler_params=pltpu.CompilerParams(needs_layout_passes=False,
                                         use_tc_tiling_on_sc=False),
    scratch_types=[
        pltpu.VMEM((_BSZ,), jnp.int32),
        pltpu.VMEM((_BSZ,), jnp.int32),
        pltpu.VMEM((2, _BSZ), jnp.float32),
        pltpu.VMEM((_BSZ,), jnp.int32),
        pltpu.VMEM((_BSZ,), jnp.int32),
        pltpu.VMEM((2, _BSZ), jnp.float32),
        pltpu.VMEM((_NT, 8), jnp.float32),
        pltpu.VMEM((_NT,), jnp.float32),
        pltpu.VMEM((_NT,), jnp.float32),
        pltpu.VMEM((_NT,), jnp.float32),
        pltpu.SemaphoreType.DMA,
        pltpu.SemaphoreType.DMA,
    ],
)(_sc_body)


_BN = 256


def _combine_body(*refs):
    planes = refs[:32]
    emb_ref, r1_ref, r2_ref, rb_ref, a_ref, b_ref = refs[32:]
    s = jnp.concatenate([p[0, 0] for p in planes], axis=1)
    f = lax.dot_general(emb_ref[...], r2_ref[...], (((0,), (0,)), ((), ())),
                        preferred_element_type=jnp.float32)
    a = jnp.dot(s, r1_ref[...], preferred_element_type=jnp.float32) * f
    a_ref[...] = a
    aa = jnp.concatenate([a, a * a], axis=1)
    b_ref[...] = jnp.dot(aa, rb_ref[...], preferred_element_type=jnp.float32)


def _combine(s2, emb_cols, r1, r2, rb):
    plane_specs = [
        pl.BlockSpec((1, 1, _BN, 8), lambda i, c=c, s=s: (c, s, i, 0))
        for c in range(2) for s in range(16)
    ]
    return pl.pallas_call(
        _combine_body,
        grid=(_NT // _BN,),
        out_shape=(
            jax.ShapeDtypeStruct((_N, 540), jnp.float32),
            jax.ShapeDtypeStruct((_N, 162), jnp.float32),
        ),
        in_specs=plane_specs + [
            pl.BlockSpec((8, _BN), lambda i: (0, i)),
            pl.BlockSpec((256, 540), lambda i: (0, 0)),
            pl.BlockSpec((8, 540), lambda i: (0, 0)),
            pl.BlockSpec((1080, 162), lambda i: (0, 0)),
        ],
        out_specs=(
            pl.BlockSpec((_BN, 540), lambda i: (i, 0)),
            pl.BlockSpec((_BN, 162), lambda i: (i, 0)),
        ),
    )(*([s2] * 32), emb_cols, r1, r2, rb)


def kernel(positions, atomic_numbers, edge_index, edge_lengths, edge_vectors,
           W):
    src = edge_index[0].astype(jnp.int32)
    dst = edge_index[1].astype(jnp.int32)
    pad_e = _EPAD - _E
    src_p = jnp.concatenate([src, jnp.zeros((pad_e,), jnp.int32)])
    dst_p = jnp.concatenate([dst, jnp.full((pad_e,), _N, jnp.int32)])

    rv = jnp.concatenate([edge_lengths.reshape(1, _E),
                          edge_vectors.T.astype(jnp.float32),
                          jnp.zeros((4, _E), jnp.float32)], axis=0)
    rv = jnp.pad(rv, ((0, 0), (0, pad_e)), constant_values=1.0)

    an8 = jnp.pad(atomic_numbers.astype(jnp.int32).reshape(1, _N),
                  ((0, 7), (0, _NT - _N)))

    emb_cols = _embed(an8, W.astype(jnp.float32))
    pt = _edgefeat(rv)
    s2 = _sc_call(pt, src_p, dst_p, emb_cols)
    node_a, node_b = _combine(s2, emb_cols,
                              jnp.asarray(_R1), jnp.asarray(_R2),
                              jnp.asarray(_RB))
    return node_a.reshape(_N, 6, 10, 9), node_b.reshape(_N, 6, 3, 9)
